# Initial kernel scaffold; baseline (speedup 1.0000x reference)
import functools

import jax
import jax.numpy as jnp
from jax import lax
from jax.experimental import pallas as pl
from jax.experimental.pallas import tpu as pltpu
from jax.experimental.pallas import tpu_sc as plsc

_N = 10000
_E = 320000
_D = 128
_NC = 2
_NS = 16
_NW = _NC * _NS
_EPT = _E // _NW
_C = 80
_NCHUNK = _EPT // _C
_RPT = _N // _NS
_ZR = 125

_INV_SQRT2 = 0.7071067811865476


def _gelu(u):
    z = jnp.abs(u) * _INV_SQRT2
    t = 1.0 / (1.0 + 0.3275911 * z)
    poly = t * (0.254829592 + t * (-0.284496736 + t * (1.421413741
                + t * (-1.453152027 + t * 1.061405429))))
    erf = jnp.sign(u) * (1.0 - poly * jnp.exp(-z * z))
    return 0.5 * u * (1.0 + erf)


_sc_mesh = plsc.VectorSubcoreMesh(core_axis_name="c", subcore_axis_name="s")


@functools.partial(
    pl.kernel,
    mesh=_sc_mesh,
    out_type=jax.ShapeDtypeStruct((_NC, _N, _D), jnp.float32),
    scratch_types=[
        pltpu.VMEM((_C,), jnp.int32),
        pltpu.VMEM((_C,), jnp.int32),
        pltpu.VMEM((_C,), jnp.int32),
        pltpu.VMEM((_C, _D), jnp.float32),
        pltpu.VMEM((_C, _D), jnp.float32),
        pltpu.VMEM((_ZR, _D), jnp.float32),
        pltpu.VMEM_SHARED((_N, _D), jnp.float32),
        pltpu.SemaphoreType.DMA,
        pltpu.SemaphoreType.DMA,
    ],
)
def _edge_kernel(h_hbm, combo_hbm, src_hbm, dst_hbm, code_hbm, out_hbm,
                 src_v, dst_v, code_v, xbuf, cbuf, zbuf, agg_sh, sem1, sem2):
    c = lax.axis_index("c")
    s = lax.axis_index("s")
    wid = c * _NS + s

    zero16 = jnp.zeros((16,), jnp.float32)

    def zrow(r, carry):
        for j in range(_D // 16):
            zbuf[r, pl.ds(j * 16, 16)] = zero16
        return carry

    lax.fori_loop(0, _ZR, zrow, 0)
    for t in range(_RPT // _ZR):
        pltpu.sync_copy(zbuf, agg_sh.at[pl.ds(s * _RPT + t * _ZR, _ZR)])
    plsc.subcore_barrier()

    ebase = wid * _EPT

    def chunk(k, carry):
        base = ebase + k * _C
        pltpu.sync_copy(src_hbm.at[pl.ds(base, _C)], src_v)
        pltpu.sync_copy(code_hbm.at[pl.ds(base, _C)], code_v)
        pltpu.sync_copy(dst_hbm.at[pl.ds(base, _C)], dst_v)
        gx = pltpu.async_copy(h_hbm.at[src_v], xbuf, sem1)
        gc = pltpu.async_copy(combo_hbm.at[code_v], cbuf, sem2)
        gx.wait()
        gc.wait()

        def row(r, rc):
            for j in range(_D // 16):
                sl = pl.ds(j * 16, 16)
                xbuf[r, sl] = _gelu(xbuf[r, sl] + cbuf[r, sl])
            return rc

        lax.fori_loop(0, _C, row, 0)
        pltpu.sync_copy(xbuf, agg_sh.at[dst_v], add=True)
        return carry

    lax.fori_loop(0, _NCHUNK, chunk, 0)

    plsc.subcore_barrier()
    pltpu.sync_copy(agg_sh.at[pl.ds(s * _RPT, _RPT)],
                    out_hbm.at[c, pl.ds(s * _RPT, _RPT)])


_NB = 10
_BR = _N // _NB


def _node_body(x_ref, agg_ref, w1_ref, b1_ref, w2_ref, b2_ref,
               gam_ref, bet_ref, eps_ref, out_ref, h2_scr, ssum, ssq):
    p = pl.program_id(0)
    b = pl.program_id(1)

    @pl.when(p == 0)
    def _pass0():
        hb = ((1.0 + eps_ref[0, 0]) * x_ref[...]
              + agg_ref[0] + agg_ref[1])
        a1 = _gelu(jnp.dot(hb, w1_ref[...],
                           preferred_element_type=jnp.float32) + b1_ref[...])
        h2 = jnp.dot(a1, w2_ref[...],
                     preferred_element_type=jnp.float32) + b2_ref[...]
        h2_scr[pl.ds(b * _BR, _BR), :] = h2
        colsum = jnp.sum(h2, axis=0, keepdims=True)
        colsq = jnp.sum(h2 * h2, axis=0, keepdims=True)

        @pl.when(b == 0)
        def _init():
            ssum[...] = colsum
            ssq[...] = colsq

        @pl.when(b != 0)
        def _acc():
            ssum[...] += colsum
            ssq[...] += colsq

    @pl.when(p == 1)
    def _pass1():
        mu = ssum[...] / _N
        var = ssq[...] / _N - mu * mu
        h2 = h2_scr[pl.ds(b * _BR, _BR), :]
        g = (h2 - mu) * gam_ref[...] * lax.rsqrt(var + 1e-5) + bet_ref[...]
        out_ref[...] = (x_ref[...] + _gelu(g)) * _INV_SQRT2


_node_call = pl.pallas_call(
    _node_body,
    grid=(2, _NB),
    in_specs=[
        pl.BlockSpec((_BR, _D), lambda p, b: (b, 0)),
        pl.BlockSpec((_NC, _BR, _D), lambda p, b: (0, b, 0)),
        pl.BlockSpec((_D, _D), lambda p, b: (0, 0)),
        pl.BlockSpec((1, _D), lambda p, b: (0, 0)),
        pl.BlockSpec((_D, _D), lambda p, b: (0, 0)),
        pl.BlockSpec((1, _D), lambda p, b: (0, 0)),
        pl.BlockSpec((1, _D), lambda p, b: (0, 0)),
        pl.BlockSpec((1, _D), lambda p, b: (0, 0)),
        pl.BlockSpec((1, 1), lambda p, b: (0, 0)),
    ],
    out_specs=pl.BlockSpec((_BR, _D), lambda p, b: (b, 0)),
    out_shape=jax.ShapeDtypeStruct((_N, _D), jnp.float32),
    scratch_shapes=[
        pltpu.VMEM((_N, _D), jnp.float32),
        pltpu.VMEM((1, _D), jnp.float32),
        pltpu.VMEM((1, _D), jnp.float32),
    ],
)


def kernel(x, edge_index, edge_attr, params):
    src = edge_index[0]
    dst = edge_index[1]
    code = edge_attr[:, 0] * 12 + edge_attr[:, 1] * 2 + edge_attr[:, 2]
    h = x
    for p in params:
        combo = (p['tab0'][:, None, None, :]
                 + p['tab1'][None, :, None, :]
                 + p['tab2'][None, None, :, :]).reshape(60, _D)
        agg2 = _edge_kernel(h, combo, src, dst, code)
        h = _node_call(h, agg2,
                       p['W1'], p['b1'].reshape(1, _D),
                       p['W2'], p['b2'].reshape(1, _D),
                       p['gamma'].reshape(1, _D), p['beta'].reshape(1, _D),
                       p['eps'].reshape(1, 1))
    return h


# speedup vs baseline: 1.8784x; 1.8784x over previous
import functools

import jax
import jax.numpy as jnp
from jax import lax
from jax.experimental import pallas as pl
from jax.experimental.pallas import tpu as pltpu
from jax.experimental.pallas import tpu_sc as plsc

_N = 10000
_E = 320000
_D = 128
_NC = 2
_NS = 16
_NW = _NC * _NS
_EPT = _E // _NW
_C = 80
_NCHUNK = _EPT // _C
_CPT = 10
_RPT = _N // _CPT
_ZR = 200

_INV_SQRT2 = 0.7071067811865476


def _gelu(u):
    z = jnp.abs(u) * _INV_SQRT2
    t = 1.0 / (1.0 + 0.3275911 * z)
    poly = t * (0.254829592 + t * (-0.284496736 + t * (1.421413741
                + t * (-1.453152027 + t * 1.061405429))))
    erf = jnp.sign(u) * (1.0 - poly * jnp.exp(-z * z))
    return 0.5 * u * (1.0 + erf)


_sc_mesh = plsc.VectorSubcoreMesh(core_axis_name="c", subcore_axis_name="s")


@functools.partial(
    pl.kernel,
    mesh=_sc_mesh,
    out_type=jax.ShapeDtypeStruct((_NC, _N, _D), jnp.float32),
    scratch_types=[
        pltpu.VMEM((_C,), jnp.int32),
        pltpu.VMEM((_C,), jnp.int32),
        pltpu.VMEM((_C,), jnp.int32),
        pltpu.VMEM((_C, _D), jnp.float32),
        pltpu.VMEM((_C, _D), jnp.float32),
        pltpu.VMEM((_ZR, _D), jnp.float32),
        pltpu.VMEM_SHARED((_N, _D), jnp.float32),
        pltpu.SemaphoreType.DMA,
        pltpu.SemaphoreType.DMA,
    ],
)
def _edge_kernel(h_hbm, combo_hbm, src_hbm, dst_hbm, code_hbm, out_hbm,
                 src_v, dst_v, code_v, xbuf, cbuf, zbuf, agg_sh, sem1, sem2):
    c = lax.axis_index("c")
    s = lax.axis_index("s")
    wid = c * _NS + s

    zero16 = jnp.zeros((16,), jnp.float32)

    def zrow(r, carry):
        for j in range(_D // 16):
            zbuf[r, pl.ds(j * 16, 16)] = zero16
        return carry

    lax.fori_loop(0, _ZR, zrow, 0)

    @pl.when(s < _CPT)
    def _zero_agg():
        for t in range(_RPT // _ZR):
            pltpu.sync_copy(zbuf, agg_sh.at[pl.ds(s * _RPT + t * _ZR, _ZR)])

    plsc.subcore_barrier()

    ebase = wid * _EPT

    def chunk(k, carry):
        base = ebase + k * _C
        pltpu.sync_copy(src_hbm.at[pl.ds(base, _C)], src_v)
        pltpu.sync_copy(code_hbm.at[pl.ds(base, _C)], code_v)
        pltpu.sync_copy(dst_hbm.at[pl.ds(base, _C)], dst_v)
        gx = pltpu.async_copy(h_hbm.at[src_v], xbuf, sem1)
        gc = pltpu.async_copy(combo_hbm.at[code_v], cbuf, sem2)
        gx.wait()
        gc.wait()

        def row(r, rc):
            for j in range(_D // 16):
                sl = pl.ds(j * 16, 16)
                xbuf[r, sl] = _gelu(xbuf[r, sl] + cbuf[r, sl])
            return rc

        lax.fori_loop(0, _C, row, 0)
        pltpu.sync_copy(xbuf, agg_sh.at[dst_v], add=True)
        return carry

    lax.fori_loop(0, _NCHUNK, chunk, 0)

    plsc.subcore_barrier()

    @pl.when(s < _CPT)
    def _copy_out():
        pltpu.sync_copy(agg_sh.at[pl.ds(s * _RPT, _RPT)],
                        out_hbm.at[c, pl.ds(s * _RPT, _RPT)])


_NB = 10
_BR = _N // _NB


def _node_body(x_ref, agg_ref, w1_ref, b1_ref, w2_ref, b2_ref,
               gam_ref, bet_ref, eps_ref, out_ref, h2_scr, ssum, ssq):
    p = pl.program_id(0)
    b = pl.program_id(1)

    @pl.when(p == 0)
    def _pass0():
        hb = ((1.0 + eps_ref[0, 0]) * x_ref[...]
              + agg_ref[0] + agg_ref[1])
        a1 = _gelu(jnp.dot(hb, w1_ref[...],
                           preferred_element_type=jnp.float32) + b1_ref[...])
        h2 = jnp.dot(a1, w2_ref[...],
                     preferred_element_type=jnp.float32) + b2_ref[...]
        h2_scr[pl.ds(b * _BR, _BR), :] = h2
        colsum = jnp.sum(h2, axis=0, keepdims=True)
        colsq = jnp.sum(h2 * h2, axis=0, keepdims=True)

        @pl.when(b == 0)
        def _init():
            ssum[...] = colsum
            ssq[...] = colsq

        @pl.when(b != 0)
        def _acc():
            ssum[...] += colsum
            ssq[...] += colsq

    @pl.when(p == 1)
    def _pass1():
        mu = ssum[...] / _N
        var = ssq[...] / _N - mu * mu
        h2 = h2_scr[pl.ds(b * _BR, _BR), :]
        g = (h2 - mu) * gam_ref[...] * lax.rsqrt(var + 1e-5) + bet_ref[...]
        out_ref[...] = (x_ref[...] + _gelu(g)) * _INV_SQRT2


_node_call = pl.pallas_call(
    _node_body,
    grid=(2, _NB),
    in_specs=[
        pl.BlockSpec((_BR, _D), lambda p, b: (b, 0)),
        pl.BlockSpec((_NC, _BR, _D), lambda p, b: (0, b, 0)),
        pl.BlockSpec((_D, _D), lambda p, b: (0, 0)),
        pl.BlockSpec((1, _D), lambda p, b: (0, 0)),
        pl.BlockSpec((_D, _D), lambda p, b: (0, 0)),
        pl.BlockSpec((1, _D), lambda p, b: (0, 0)),
        pl.BlockSpec((1, _D), lambda p, b: (0, 0)),
        pl.BlockSpec((1, _D), lambda p, b: (0, 0)),
        pl.BlockSpec((1, 1), lambda p, b: (0, 0)),
    ],
    out_specs=pl.BlockSpec((_BR, _D), lambda p, b: (b, 0)),
    out_shape=jax.ShapeDtypeStruct((_N, _D), jnp.float32),
    scratch_shapes=[
        pltpu.VMEM((_N, _D), jnp.float32),
        pltpu.VMEM((1, _D), jnp.float32),
        pltpu.VMEM((1, _D), jnp.float32),
    ],
)


def kernel(x, edge_index, edge_attr, params):
    src = edge_index[0]
    dst = edge_index[1]
    code = edge_attr[:, 0] * 12 + edge_attr[:, 1] * 2 + edge_attr[:, 2]
    h = x
    for p in params:
        combo = (p['tab0'][:, None, None, :]
                 + p['tab1'][None, :, None, :]
                 + p['tab2'][None, None, :, :]).reshape(60, _D)
        agg2 = _edge_kernel(h, combo, src, dst, code)
        h = _node_call(h, agg2,
                       p['W1'], p['b1'].reshape(1, _D),
                       p['W2'], p['b2'].reshape(1, _D),
                       p['gamma'].reshape(1, _D), p['beta'].reshape(1, _D),
                       p['eps'].reshape(1, 1))
    return h
